# Initial kernel scaffold; baseline (speedup 1.0000x reference)
import jax
import jax.numpy as jnp
from jax.experimental import pallas as pl


def kernel(x, edge_index, edge_attr, batch, W1, b1, g1, be1, W2, b2, g2, be2, pw, W3, b3, g3, be3, fw1, fb1, fw2, fb2):
    raise NotImplementedError("write your pallas kernel here")


# speedup vs baseline: 14.0005x; 14.0005x over previous
import functools

import jax
import jax.numpy as jnp
from jax import lax
from jax.experimental import pallas as pl
from jax.experimental.pallas import tpu as pltpu
from jax.experimental.pallas import tpu_sc as plsc

N = 10000
E = 320000
C_IN = 128
H = 64
K = 5000
F = 128

NC = 2
NS = 16
NW = NC * NS
CH = 128
E_PAD = 327680
PER_TILE = E_PAD // NW
CHUNKS = PER_TILE // CH
N_PAD = 10240
SLICE = N_PAD // NS

_INT_MIN = -2147483648


_MESH = plsc.VectorSubcoreMesh(core_axis_name="c", subcore_axis_name="s")


def _bcast_lane(v16, lane):
    idx = jnp.full((16,), lane, jnp.int32)
    return v16.at[idx].get(mode="promise_in_bounds")


@functools.partial(
    pl.kernel,
    out_type=jax.ShapeDtypeStruct((NC, N_PAD, F), jnp.float32),
    mesh=_MESH,
    scratch_types=[
        pltpu.VMEM((CH,), jnp.int32),
        pltpu.VMEM((CH,), jnp.float32),
        pltpu.VMEM((16,), jnp.float32),
        pltpu.VMEM((CH, F), jnp.float32),
        pltpu.VMEM_SHARED((N_PAD, F), jnp.float32),
    ],
)
def _sc_wdeg(colp, ewp, onesv, zeros, out, colbuf, ewbuf, onesb, rows, accsh):
    cid = lax.axis_index("c")
    sid = lax.axis_index("s")
    wid = cid * NS + sid
    pltpu.sync_copy(zeros.at[pl.ds(sid * SLICE, SLICE)],
                    accsh.at[pl.ds(sid * SLICE, SLICE)])
    pltpu.sync_copy(zeros.at[pl.ds(0, CH)], rows)
    pltpu.sync_copy(onesv, onesb)
    plsc.subcore_barrier()
    base = wid * PER_TILE

    def chunk_body(ci, _):
        off = base + ci * CH
        pltpu.sync_copy(colp.at[pl.ds(off, CH)], colbuf)
        pltpu.sync_copy(ewp.at[pl.ds(off, CH)], ewbuf)

        for g in range(CH // 16):
            w16 = ewbuf[pl.ds(g * 16, 16)]
            one = onesb[pl.ds(0, 16)]
            for l in range(16):
                rows[g * 16 + l, pl.ds(0, 16)] = one * _bcast_lane(w16, l)
        pltpu.sync_copy(rows, accsh.at[colbuf], add=True)
        return 0

    lax.fori_loop(0, CHUNKS, chunk_body, 0)
    plsc.subcore_barrier()
    pltpu.sync_copy(accsh.at[pl.ds(sid * SLICE, SLICE)],
                    out.at[cid, pl.ds(sid * SLICE, SLICE)])


@functools.partial(
    pl.kernel,
    out_type=jax.ShapeDtypeStruct((NC, N_PAD, F), jnp.float32),
    mesh=_MESH,
    scratch_types=[
        pltpu.VMEM((CH,), jnp.int32),
        pltpu.VMEM((CH,), jnp.int32),
        pltpu.VMEM((CH,), jnp.float32),
        pltpu.VMEM((CH, F), jnp.float32),
        pltpu.VMEM_SHARED((N_PAD, F), jnp.float32),
        pltpu.SemaphoreType.DMA,
    ],
)
def _sc_acc(rowp, colp, ewp, gt, zeros, out, rowbuf, colbuf, ewbuf, rows,
            accsh, sem):
    cid = lax.axis_index("c")
    sid = lax.axis_index("s")
    wid = cid * NS + sid
    pltpu.sync_copy(zeros.at[pl.ds(sid * SLICE, SLICE)],
                    accsh.at[pl.ds(sid * SLICE, SLICE)])
    plsc.subcore_barrier()
    base = wid * PER_TILE

    def chunk_body(ci, _):
        off = base + ci * CH
        pltpu.sync_copy(rowp.at[pl.ds(off, CH)], rowbuf)
        pltpu.sync_copy(colp.at[pl.ds(off, CH)], colbuf)
        pltpu.sync_copy(ewp.at[pl.ds(off, CH)], ewbuf)
        pltpu.async_copy(gt.at[rowbuf], rows, sem).wait()

        for g in range(CH // 16):
            w16 = ewbuf[pl.ds(g * 16, 16)]
            for l in range(16):
                wb = _bcast_lane(w16, l)
                e = g * 16 + l
                for q in range(F // 16):
                    rows[e, pl.ds(q * 16, 16)] = (
                        rows[e, pl.ds(q * 16, 16)] * wb)
        pltpu.sync_copy(rows, accsh.at[colbuf], add=True)
        return 0

    lax.fori_loop(0, CHUNKS, chunk_body, 0)
    plsc.subcore_barrier()
    pltpu.sync_copy(accsh.at[pl.ds(sid * SLICE, SLICE)],
                    out.at[cid, pl.ds(sid * SLICE, SLICE)])


def _tc_call(body, out_shapes):
    return pl.pallas_call(body, out_shape=out_shapes)


def _bn_relu(out, g, be):
    m = jnp.mean(out, axis=0, keepdims=True)
    v = jnp.mean((out - m) ** 2, axis=0, keepdims=True)
    return jax.nn.relu((out - m) * lax.rsqrt(v + 1e-5) * g[None, :] + be[None, :])


def _pad_right(a):
    return jnp.concatenate([a, jnp.zeros((N, F - H), jnp.float32)], axis=1)


def _tc_pre_body(x_ref, w1_ref, degp_ref, h1in_ref, gt1_ref, dis_ref, inv_ref):
    deg = degp_ref[0, :N, 0:1] + degp_ref[1, :N, 0:1] + 1.0
    dis = lax.rsqrt(deg)
    inv = 1.0 / deg
    h1in = jnp.dot(x_ref[...], w1_ref[...], preferred_element_type=jnp.float32)
    h1in_ref[...] = h1in
    gt1_ref[...] = _pad_right(dis * h1in)
    dis_ref[...] = dis
    inv_ref[...] = inv


def _tc_layer2_body(accp_ref, h1in_ref, dis_ref, inv_ref, b1_ref, g1_ref,
                    be1_ref, w2_ref, h2in_ref, gt2_ref):
    acc = accp_ref[0, :N, :H] + accp_ref[1, :N, :H]
    out1 = dis_ref[...] * acc + inv_ref[...] * h1in_ref[...] + b1_ref[...][None, :]
    h1 = _bn_relu(out1, g1_ref[...], be1_ref[...])
    h2in = jnp.dot(h1, w2_ref[...], preferred_element_type=jnp.float32)
    h2in_ref[...] = h2in
    gt2_ref[...] = _pad_right(dis_ref[...] * h2in)


def _tc_score_body(accp_ref, h2in_ref, dis_ref, inv_ref, b2_ref, g2_ref,
                   be2_ref, pw_ref, h2_ref, s_ref, key_ref):
    acc = accp_ref[0, :N, :H] + accp_ref[1, :N, :H]
    out2 = dis_ref[...] * acc + inv_ref[...] * h2in_ref[...] + b2_ref[...][None, :]
    h2 = _bn_relu(out2, g2_ref[...], be2_ref[...])
    h2_ref[...] = h2
    pw = pw_ref[...]
    rn = lax.rsqrt(jnp.sum(pw * pw))
    s = jnp.tanh(jnp.dot(h2, pw[:, None], preferred_element_type=jnp.float32) * rn)
    s_ref[...] = s
    i = lax.bitcast_convert_type(s, jnp.int32)
    key_ref[...] = jnp.where(i < 0, i ^ jnp.int32(0x7FFFFFFF), i)


def _tc_select_body(keyq_ref, keepq_ref):
    keyq = keyq_ref[...]
    cnt_pos = jnp.sum((keyq >= 0).astype(jnp.int32))
    t0 = jnp.where(cnt_pos >= K, jnp.int32(0), jnp.int32(_INT_MIN))

    def bit_body(bi, t):
        t_try = t | (jnp.int32(1) << (30 - bi))
        c = jnp.sum((keyq >= t_try).astype(jnp.int32))
        return jnp.where(c >= K, t_try, t)

    t = lax.fori_loop(0, 31, bit_body, t0)
    n_gt = jnp.sum((keyq > t).astype(jnp.int32))
    need = (K - n_gt).astype(jnp.float32)
    tie = (keyq == t).astype(jnp.float32)
    rowsum = jnp.sum(tie, axis=1, keepdims=True)
    r_i = lax.broadcasted_iota(jnp.int32, (80, 80), 0)
    r_j = lax.broadcasted_iota(jnp.int32, (80, 80), 1)
    sl = (r_j < r_i).astype(jnp.float32)
    rowoff = jnp.dot(sl, rowsum, preferred_element_type=jnp.float32)
    c_i = lax.broadcasted_iota(jnp.int32, (128, 128), 0)
    c_j = lax.broadcasted_iota(jnp.int32, (128, 128), 1)
    su = (c_i < c_j).astype(jnp.float32)
    within = jnp.dot(tie, su, preferred_element_type=jnp.float32)
    rank = rowoff + within
    keepq = jnp.where(keyq > t, 1.0, 0.0) + tie * (rank < need).astype(jnp.float32)
    keepq_ref[...] = keepq


def _tc_keep128_body(keep_ref, k128_ref):
    k128_ref[...] = jnp.broadcast_to(keep_ref[...], (N, F))


def _tc_pool_body(h2_ref, s_ref, keep_ref, d3p_ref, w3_ref, h3in_ref,
                  gt3_ref, dis3_ref, inv3_ref):
    keep = keep_ref[...]
    hp = h2_ref[...] * s_ref[...] * keep
    h3in = jnp.dot(hp, w3_ref[...], preferred_element_type=jnp.float32)
    deg3 = d3p_ref[0, :N, 0:1] + d3p_ref[1, :N, 0:1] + keep
    safe = jnp.where(deg3 > 0, deg3, 1.0)
    dis3 = keep * lax.rsqrt(safe)
    inv3 = keep / safe
    h3in_ref[...] = h3in
    gt3_ref[...] = dis3 * h3in
    dis3_ref[...] = dis3
    inv3_ref[...] = inv3


def _tc_final_body(accp_ref, h3in_ref, dis3_ref, inv3_ref, keep_ref, b3_ref,
                   g3_ref, be3_ref, fw1_ref, fb1_ref, fw2_ref, fb2_ref, o_ref):
    keep = keep_ref[...]
    acc = accp_ref[0, :N, :] + accp_ref[1, :N, :]
    out3 = dis3_ref[...] * acc + inv3_ref[...] * h3in_ref[...] + b3_ref[...][None, :]
    kf = jnp.float32(K)
    m = jnp.sum(keep * out3, axis=0, keepdims=True) / kf
    v = jnp.sum(keep * (out3 - m) ** 2, axis=0, keepdims=True) / kf
    h3 = jax.nn.relu((out3 - m) * lax.rsqrt(v + 1e-5) * g3_ref[...][None, :]
                     + be3_ref[...][None, :])
    gm = jnp.sum(keep * h3, axis=0, keepdims=True) / kf
    gx = jnp.max(jnp.where(keep > 0, h3, -3.4e38), axis=0, keepdims=True)
    emb = jnp.concatenate([gm, gx], axis=1)
    z = jax.nn.relu(jnp.dot(emb, fw1_ref[...], preferred_element_type=jnp.float32)
                    + fb1_ref[...][None, :])
    o_ref[...] = (jnp.dot(z, fw2_ref[...], preferred_element_type=jnp.float32)
                  + fb2_ref[...][None, :])


def kernel(x, edge_index, edge_attr, batch, W1, b1, g1, be1, W2, b2, g2, be2,
           pw, W3, b3, g3, be3, fw1, fb1, fw2, fb2):
    f32 = jnp.float32
    pad = E_PAD - E
    pad_idx = jnp.arange(pad, dtype=jnp.int32) % N
    rowp = jnp.concatenate([edge_index[0], pad_idx])
    colp = jnp.concatenate([edge_index[1], pad_idx])
    ewp = jnp.concatenate([edge_attr, jnp.zeros((pad,), f32)])
    zeros = jnp.zeros((N_PAD, F), f32)
    onesv = jnp.ones((16,), f32)

    degp = _sc_wdeg(colp, ewp, onesv, zeros)

    sd = jax.ShapeDtypeStruct
    h1in, gt1, dis, inv = _tc_call(
        _tc_pre_body,
        [sd((N, H), f32), sd((N, F), f32), sd((N, 1), f32), sd((N, 1), f32)],
    )(x, W1, degp)

    acc1p = _sc_acc(rowp, colp, ewp, gt1, zeros)

    h2in, gt2 = _tc_call(
        _tc_layer2_body, [sd((N, H), f32), sd((N, F), f32)],
    )(acc1p, h1in, dis, inv, b1, g1, be1, W2)

    acc2p = _sc_acc(rowp, colp, ewp, gt2, zeros)

    h2, s, key = _tc_call(
        _tc_score_body, [sd((N, H), f32), sd((N, 1), f32), sd((N, 1), jnp.int32)],
    )(acc2p, h2in, dis, inv, b2, g2, be2, pw)

    keyq = jnp.concatenate(
        [key[:, 0], jnp.full((80 * 128 - N,), _INT_MIN, jnp.int32)]).reshape(80, 128)
    keepq = _tc_call(_tc_select_body, sd((80, 128), f32))(keyq)
    keep = keepq.reshape(80 * 128)[:N, None]

    keep128 = _tc_call(_tc_keep128_body, sd((N, F), f32))(keep)
    d3p = _sc_acc(rowp, colp, ewp, keep128, zeros)

    h3in, gt3, dis3, inv3 = _tc_call(
        _tc_pool_body,
        [sd((N, F), f32), sd((N, F), f32), sd((N, 1), f32), sd((N, 1), f32)],
    )(h2, s, keep, d3p, W3)

    acc3p = _sc_acc(rowp, colp, ewp, gt3, zeros)

    out = _tc_call(_tc_final_body, sd((1, 2), f32))(
        acc3p, h3in, dis3, inv3, keep, b3, g3, be3, fw1, fb1, fw2, fb2)
    return out


# speedup vs baseline: 16.3530x; 1.1680x over previous
import functools

import jax
import jax.numpy as jnp
from jax import lax
from jax.experimental import pallas as pl
from jax.experimental.pallas import tpu as pltpu
from jax.experimental.pallas import tpu_sc as plsc

N = 10000
E = 320000
C_IN = 128
H = 64
K = 5000

NC = 2
NS = 16
NW = NC * NS
CH = 128
E_PAD = 327680
PER_TILE = E_PAD // NW
CHUNKS = PER_TILE // CH
N_PAD = 10240
SLICE = N_PAD // NS

_INT_MIN = -2147483648


_MESH = plsc.VectorSubcoreMesh(core_axis_name="c", subcore_axis_name="s")


def _bcast_lane(v16, lane):
    idx = jnp.full((16,), lane, jnp.int32)
    return v16.at[idx].get(mode="promise_in_bounds")


def _make_sc_acc(W):
    @functools.partial(
        pl.kernel,
        out_type=jax.ShapeDtypeStruct((NC, N_PAD, W), jnp.float32),
        mesh=_MESH,
        scratch_types=[
            pltpu.VMEM((CH,), jnp.int32),
            pltpu.VMEM((CH,), jnp.int32),
            pltpu.VMEM((CH,), jnp.int32),
            pltpu.VMEM((CH,), jnp.int32),
            pltpu.VMEM((CH,), jnp.float32),
            pltpu.VMEM((CH,), jnp.float32),
            pltpu.VMEM((CH, W), jnp.float32),
            pltpu.VMEM((CH, W), jnp.float32),
            pltpu.VMEM_SHARED((N_PAD, W), jnp.float32),
            pltpu.SemaphoreType.DMA,
            pltpu.SemaphoreType.DMA,
        ],
    )
    def _sc_acc(rowp, colp, ewp, gt, zeros, out, rowb0, rowb1, colb0, colb1,
                ewb0, ewb1, rows0, rows1, accsh, sem0, sem1):
        cid = lax.axis_index("c")
        sid = lax.axis_index("s")
        wid = cid * NS + sid
        pltpu.sync_copy(zeros.at[pl.ds(sid * SLICE, SLICE)],
                        accsh.at[pl.ds(sid * SLICE, SLICE)])
        plsc.subcore_barrier()
        base = wid * PER_TILE

        rowb = [rowb0, rowb1]
        colb = [colb0, colb1]
        ewb = [ewb0, ewb1]
        rows = [rows0, rows1]
        sems = [sem0, sem1]

        def fire(ci, b):
            off = base + ci * CH
            pltpu.sync_copy(rowp.at[pl.ds(off, CH)], rowb[b])
            pltpu.sync_copy(colp.at[pl.ds(off, CH)], colb[b])
            pltpu.sync_copy(ewp.at[pl.ds(off, CH)], ewb[b])
            pltpu.async_copy(gt.at[rowb[b]], rows[b], sems[b])

        fire(0, 0)

        def pair_body(gi, _):
            for b in range(2):
                ci = gi * 2 + b

                @pl.when(ci + 1 < CHUNKS)
                def _():
                    fire(ci + 1, 1 - b)

                pltpu.make_async_copy(gt.at[rowb[b]], rows[b], sems[b]).wait()
                for g in range(CH // 16):
                    w16 = ewb[b][pl.ds(g * 16, 16)]
                    for l in range(16):
                        wb = _bcast_lane(w16, l)
                        e = g * 16 + l
                        for q in range(W // 16):
                            rows[b][e, pl.ds(q * 16, 16)] = (
                                rows[b][e, pl.ds(q * 16, 16)] * wb)
                pltpu.sync_copy(rows[b], accsh.at[colb[b]], add=True)
            return 0

        lax.fori_loop(0, CHUNKS // 2, pair_body, 0)
        plsc.subcore_barrier()
        pltpu.sync_copy(accsh.at[pl.ds(sid * SLICE, SLICE)],
                        out.at[cid, pl.ds(sid * SLICE, SLICE)])

    return _sc_acc


_sc_acc128 = _make_sc_acc(128)


@functools.partial(
    pl.kernel,
    out_type=jax.ShapeDtypeStruct((NC, N_PAD, 16), jnp.float32),
    mesh=_MESH,
    scratch_types=[
        pltpu.VMEM((CH,), jnp.int32),
        pltpu.VMEM((CH,), jnp.float32),
        pltpu.VMEM((16,), jnp.float32),
        pltpu.VMEM((CH, 16), jnp.float32),
        pltpu.VMEM_SHARED((N_PAD, 16), jnp.float32),
    ],
)
def _sc_wdeg(colp, ewp, onesv, zeros, out, colbuf, ewbuf, onesb, rows, accsh):
    cid = lax.axis_index("c")
    sid = lax.axis_index("s")
    wid = cid * NS + sid
    pltpu.sync_copy(zeros.at[pl.ds(sid * SLICE, SLICE)],
                    accsh.at[pl.ds(sid * SLICE, SLICE)])
    pltpu.sync_copy(onesv, onesb)
    plsc.subcore_barrier()
    base = wid * PER_TILE

    def chunk_body(ci, _):
        off = base + ci * CH
        pltpu.sync_copy(colp.at[pl.ds(off, CH)], colbuf)
        pltpu.sync_copy(ewp.at[pl.ds(off, CH)], ewbuf)

        for g in range(CH // 16):
            w16 = ewbuf[pl.ds(g * 16, 16)]
            one = onesb[pl.ds(0, 16)]
            for l in range(16):
                rows[g * 16 + l, pl.ds(0, 16)] = one * _bcast_lane(w16, l)
        pltpu.sync_copy(rows, accsh.at[colbuf], add=True)
        return 0

    lax.fori_loop(0, CHUNKS, chunk_body, 0)
    plsc.subcore_barrier()
    pltpu.sync_copy(accsh.at[pl.ds(sid * SLICE, SLICE)],
                    out.at[cid, pl.ds(sid * SLICE, SLICE)])


def _tc_call(body, out_shapes):
    return pl.pallas_call(body, out_shape=out_shapes)


def _bn_relu(out, g, be):
    m = jnp.mean(out, axis=0, keepdims=True)
    v = jnp.mean((out - m) ** 2, axis=0, keepdims=True)
    return jax.nn.relu((out - m) * lax.rsqrt(v + 1e-5) * g[None, :] + be[None, :])


def _pad_right(a):
    return jnp.concatenate([a, jnp.zeros((N, 128 - H), jnp.float32)], axis=1)


def _tc_pre_body(x_ref, w1_ref, degp_ref, h1in_ref, gt1_ref, dis_ref, inv_ref):
    deg = degp_ref[0, :N, 0:1] + degp_ref[1, :N, 0:1] + 1.0
    dis = lax.rsqrt(deg)
    inv = 1.0 / deg
    h1in = jnp.dot(x_ref[...], w1_ref[...], preferred_element_type=jnp.float32)
    h1in_ref[...] = h1in
    gt1_ref[...] = _pad_right(dis * h1in)
    dis_ref[...] = dis
    inv_ref[...] = inv


def _tc_layer2_body(accp_ref, h1in_ref, dis_ref, inv_ref, b1_ref, g1_ref,
                    be1_ref, w2_ref, h2in_ref, gt2_ref):
    acc = accp_ref[0, :N, :H] + accp_ref[1, :N, :H]
    out1 = dis_ref[...] * acc + inv_ref[...] * h1in_ref[...] + b1_ref[...][None, :]
    h1 = _bn_relu(out1, g1_ref[...], be1_ref[...])
    h2in = jnp.dot(h1, w2_ref[...], preferred_element_type=jnp.float32)
    h2in_ref[...] = h2in
    gt2_ref[...] = _pad_right(dis_ref[...] * h2in)


def _tc_score_body(accp_ref, h2in_ref, dis_ref, inv_ref, b2_ref, g2_ref,
                   be2_ref, pw_ref, h2_ref, s_ref, key_ref):
    acc = accp_ref[0, :N, :H] + accp_ref[1, :N, :H]
    out2 = dis_ref[...] * acc + inv_ref[...] * h2in_ref[...] + b2_ref[...][None, :]
    h2 = _bn_relu(out2, g2_ref[...], be2_ref[...])
    h2_ref[...] = h2
    pw = pw_ref[...]
    rn = lax.rsqrt(jnp.sum(pw * pw))
    s = jnp.tanh(jnp.dot(h2, pw[:, None], preferred_element_type=jnp.float32) * rn)
    s_ref[...] = s
    i = lax.bitcast_convert_type(s, jnp.int32)
    key_ref[...] = jnp.where(i < 0, i ^ jnp.int32(0x7FFFFFFF), i)


def _tc_select_body(keyq_ref, keepq_ref):
    keyq = keyq_ref[...]
    cnt_pos = jnp.sum((keyq >= 0).astype(jnp.int32))
    t0 = jnp.where(cnt_pos >= K, jnp.int32(0), jnp.int32(_INT_MIN))

    def bit_body(bi, t):
        t_try = t | (jnp.int32(1) << (30 - bi))
        c = jnp.sum((keyq >= t_try).astype(jnp.int32))
        return jnp.where(c >= K, t_try, t)

    t = lax.fori_loop(0, 31, bit_body, t0)
    n_gt = jnp.sum((keyq > t).astype(jnp.int32))
    need = (K - n_gt).astype(jnp.float32)
    tie = (keyq == t).astype(jnp.float32)
    rowsum = jnp.sum(tie, axis=1, keepdims=True)
    r_i = lax.broadcasted_iota(jnp.int32, (80, 80), 0)
    r_j = lax.broadcasted_iota(jnp.int32, (80, 80), 1)
    sl = (r_j < r_i).astype(jnp.float32)
    rowoff = jnp.dot(sl, rowsum, preferred_element_type=jnp.float32)
    c_i = lax.broadcasted_iota(jnp.int32, (128, 128), 0)
    c_j = lax.broadcasted_iota(jnp.int32, (128, 128), 1)
    su = (c_i < c_j).astype(jnp.float32)
    within = jnp.dot(tie, su, preferred_element_type=jnp.float32)
    rank = rowoff + within
    keepq = jnp.where(keyq > t, 1.0, 0.0) + tie * (rank < need).astype(jnp.float32)
    keepq_ref[...] = keepq


def _tc_keep128_body(keep_ref, k128_ref):
    k128_ref[...] = jnp.broadcast_to(keep_ref[...], (N, 128))


def _tc_pool_body(h2_ref, s_ref, keep_ref, d3p_ref, w3_ref, h3in_ref,
                  gt3_ref, dis3_ref, inv3_ref):
    keep = keep_ref[...]
    hp = h2_ref[...] * s_ref[...] * keep
    h3in = jnp.dot(hp, w3_ref[...], preferred_element_type=jnp.float32)
    deg3 = d3p_ref[0, :N, 0:1] + d3p_ref[1, :N, 0:1] + keep
    safe = jnp.where(deg3 > 0, deg3, 1.0)
    dis3 = keep * lax.rsqrt(safe)
    inv3 = keep / safe
    h3in_ref[...] = h3in
    gt3_ref[...] = dis3 * h3in
    dis3_ref[...] = dis3
    inv3_ref[...] = inv3


def _tc_final_body(accp_ref, h3in_ref, dis3_ref, inv3_ref, keep_ref, b3_ref,
                   g3_ref, be3_ref, fw1_ref, fb1_ref, fw2_ref, fb2_ref, o_ref):
    keep = keep_ref[...]
    acc = accp_ref[0, :N, :] + accp_ref[1, :N, :]
    out3 = dis3_ref[...] * acc + inv3_ref[...] * h3in_ref[...] + b3_ref[...][None, :]
    kf = jnp.float32(K)
    m = jnp.sum(keep * out3, axis=0, keepdims=True) / kf
    v = jnp.sum(keep * (out3 - m) ** 2, axis=0, keepdims=True) / kf
    h3 = jax.nn.relu((out3 - m) * lax.rsqrt(v + 1e-5) * g3_ref[...][None, :]
                     + be3_ref[...][None, :])
    gm = jnp.sum(keep * h3, axis=0, keepdims=True) / kf
    gx = jnp.max(jnp.where(keep > 0, h3, -3.4e38), axis=0, keepdims=True)
    emb = jnp.concatenate([gm, gx], axis=1)
    z = jax.nn.relu(jnp.dot(emb, fw1_ref[...], preferred_element_type=jnp.float32)
                    + fb1_ref[...][None, :])
    o_ref[...] = (jnp.dot(z, fw2_ref[...], preferred_element_type=jnp.float32)
                  + fb2_ref[...][None, :])


def kernel(x, edge_index, edge_attr, batch, W1, b1, g1, be1, W2, b2, g2, be2,
           pw, W3, b3, g3, be3, fw1, fb1, fw2, fb2):
    f32 = jnp.float32
    pad = E_PAD - E
    pad_idx = jnp.arange(pad, dtype=jnp.int32) % N
    rowp = jnp.concatenate([edge_index[0], pad_idx])
    colp = jnp.concatenate([edge_index[1], pad_idx])
    ewp = jnp.concatenate([edge_attr, jnp.zeros((pad,), f32)])
    zeros16 = jnp.zeros((N_PAD, 16), f32)
    zeros128 = jnp.zeros((N_PAD, 128), f32)
    onesv = jnp.ones((16,), f32)

    degp = _sc_wdeg(colp, ewp, onesv, zeros16)

    sd = jax.ShapeDtypeStruct
    h1in, gt1, dis, inv = _tc_call(
        _tc_pre_body,
        [sd((N, H), f32), sd((N, 128), f32), sd((N, 1), f32), sd((N, 1), f32)],
    )(x, W1, degp)

    acc1p = _sc_acc128(rowp, colp, ewp, gt1, zeros128)

    h2in, gt2 = _tc_call(
        _tc_layer2_body, [sd((N, H), f32), sd((N, 128), f32)],
    )(acc1p, h1in, dis, inv, b1, g1, be1, W2)

    acc2p = _sc_acc128(rowp, colp, ewp, gt2, zeros128)

    h2, s, key = _tc_call(
        _tc_score_body, [sd((N, H), f32), sd((N, 1), f32), sd((N, 1), jnp.int32)],
    )(acc2p, h2in, dis, inv, b2, g2, be2, pw)

    keyq = jnp.concatenate(
        [key[:, 0], jnp.full((80 * 128 - N,), _INT_MIN, jnp.int32)]).reshape(80, 128)
    keepq = _tc_call(_tc_select_body, sd((80, 128), f32))(keyq)
    keep = keepq.reshape(80 * 128)[:N, None]

    keep128 = _tc_call(_tc_keep128_body, sd((N, 128), f32))(keep)
    d3p = _sc_acc128(rowp, colp, ewp, keep128, zeros128)

    h3in, gt3, dis3, inv3 = _tc_call(
        _tc_pool_body,
        [sd((N, 2 * H), f32), sd((N, 2 * H), f32), sd((N, 1), f32), sd((N, 1), f32)],
    )(h2, s, keep, d3p, W3)

    acc3p = _sc_acc128(rowp, colp, ewp, gt3, zeros128)

    out = _tc_call(_tc_final_body, sd((1, 2), f32))(
        acc3p, h3in, dis3, inv3, keep, b3, g3, be3, fw1, fb1, fw2, fb2)
    return out
